# baseline (speedup 1.0000x reference)
import functools

import jax
import jax.numpy as jnp
from jax import lax
from jax.experimental import pallas as pl
from jax.experimental.pallas import tpu as pltpu
from jax.experimental.pallas import tpu_sc as plsc

N = 10000
E = 320000
D = 128

NC = 2
NS = 16
NW = NC * NS
C = 80
NCH = 125
EP = NCH * C
NP = 10240
PT = NP // NS
RBT = 1000
CB = 25
NBLK = NCH // CB

_mesh = plsc.VectorSubcoreMesh(core_axis_name="c", subcore_axis_name="s")


@functools.partial(
    pl.kernel,
    out_type=jax.ShapeDtypeStruct((NC, 2, NP, 16), jnp.float32),
    mesh=_mesh,
    scratch_types=[
        pltpu.VMEM((CB, C), jnp.int32),
        pltpu.VMEM((CB, C), jnp.int32),
        pltpu.VMEM((C, 16), jnp.float32),
        pltpu.VMEM((128, 16), jnp.float32),
        pltpu.VMEM_SHARED((NP, 16), jnp.float32),
        pltpu.VMEM_SHARED((NP, 16), jnp.float32),
        pltpu.SemaphoreType.DMA,
        pltpu.SemaphoreType.DMA,
    ],
    compiler_params=pltpu.CompilerParams(use_tc_tiling_on_sc=False),
)
def _deg_kernel(ei_hbm, odeg_hbm,
                sidx, didx, ones, zb, dsrc_sh, ddst_sh, sem0, sem1):
    c = lax.axis_index("c")
    s = lax.axis_index("s")
    wid = s * NC + c

    one_hot = jnp.where(lax.iota(jnp.int32, 16) == 0, 1.0, 0.0).astype(jnp.float32)
    zvec = jnp.zeros((16,), jnp.float32)

    def _fill_ones(i, carry):
        ones[i, :] = one_hot
        return carry

    lax.fori_loop(0, C, _fill_ones, 0)

    def _fill_z(i, carry):
        zb[i, :] = zvec
        return carry

    lax.fori_loop(0, 128, _fill_z, 0)

    base = s * PT
    for k in range(PT // 128):
        pltpu.sync_copy(zb, dsrc_sh.at[pl.ds(base + k * 128, 128)])
        pltpu.sync_copy(zb, ddst_sh.at[pl.ds(base + k * 128, 128)])
    plsc.subcore_barrier()

    for blk in range(NBLK):
        pltpu.sync_copy(ei_hbm.at[0, wid, blk], sidx)
        pltpu.sync_copy(ei_hbm.at[1, wid, blk], didx)

        def _fire(j, carry):
            pltpu.async_copy(ones, dsrc_sh.at[sidx.at[j]], sem0, add=True)
            pltpu.async_copy(ones, ddst_sh.at[didx.at[j]], sem1, add=True)
            return carry

        lax.fori_loop(0, CB, _fire, 0)

        def _drain(j, carry):
            pltpu.make_async_copy(ones, dsrc_sh.at[sidx.at[j]], sem0).wait()
            pltpu.make_async_copy(ones, ddst_sh.at[didx.at[j]], sem1).wait()
            return carry

        lax.fori_loop(0, CB, _drain, 0)
    plsc.subcore_barrier()

    pltpu.sync_copy(dsrc_sh.at[pl.ds(base, PT)], odeg_hbm.at[c, 0, pl.ds(base, PT)])
    pltpu.sync_copy(ddst_sh.at[pl.ds(base, PT)], odeg_hbm.at[c, 1, pl.ds(base, PT)])


@functools.partial(
    pl.kernel,
    out_type=jax.ShapeDtypeStruct((NC, NP, D), jnp.float32),
    mesh=_mesh,
    scratch_types=[
        pltpu.VMEM((CB, C), jnp.int32),
        pltpu.VMEM((CB, C), jnp.int32),
        pltpu.VMEM((4, C, D), jnp.float32),
        pltpu.VMEM_SHARED((NP, D), jnp.float32),
        pltpu.SemaphoreType.DMA,
        pltpu.SemaphoreType.DMA,
        pltpu.SemaphoreType.DMA,
        pltpu.SemaphoreType.DMA,
        pltpu.SemaphoreType.DMA,
        pltpu.SemaphoreType.DMA,
        pltpu.SemaphoreType.DMA,
        pltpu.SemaphoreType.DMA,
    ],
)
def _agg_kernel(h_hbm, ei_hbm, out_hbm,
                sidx, didx, bufs, acc_sh,
                g0, g1, g2, g3, s0, s1, s2, s3):
    c = lax.axis_index("c")
    s = lax.axis_index("s")
    wid = s * NC + c
    semg = (g0, g1, g2, g3)
    sems = (s0, s1, s2, s3)

    zvec = jnp.zeros((16,), jnp.float32)

    def _fill_z(i, carry):
        for k in range(D // 16):
            bufs[0, i, pl.ds(k * 16, 16)] = zvec
        return carry

    lax.fori_loop(0, C, _fill_z, 0)

    base = s * PT
    for k in range(PT // C):
        pltpu.sync_copy(bufs.at[0], acc_sh.at[pl.ds(base + k * C, C)])
    plsc.subcore_barrier()

    def _wait_g(j, p):
        pltpu.make_async_copy(h_hbm.at[sidx.at[j]], bufs.at[p], semg[p]).wait()

    def _wait_s(p):
        pltpu.make_async_copy(bufs.at[p], acc_sh.at[didx.at[0]], sems[p]).wait()

    for blk in range(NBLK):
        pltpu.sync_copy(ei_hbm.at[0, wid, blk], sidx)
        pltpu.sync_copy(ei_hbm.at[1, wid, blk], didx)

        for p in range(3):
            pltpu.async_copy(h_hbm.at[sidx.at[p]], bufs.at[p], semg[p])
        _wait_g(0, 0)
        pltpu.async_copy(bufs.at[0], acc_sh.at[didx.at[0]], sems[0], add=True)
        pltpu.async_copy(h_hbm.at[sidx.at[3]], bufs.at[3], semg[3])
        for j in range(1, 4):
            _wait_g(j, j)
            pltpu.async_copy(bufs.at[j], acc_sh.at[didx.at[j]], sems[j],
                             add=True)
            _wait_s(j - 1)
            pltpu.async_copy(h_hbm.at[sidx.at[j + 3]], bufs.at[j - 1],
                             semg[j - 1])

        def _body(t, carry):
            for p in range(4):
                j = t * 4 + p
                _wait_g(j, p)
                pltpu.async_copy(bufs.at[p], acc_sh.at[didx.at[j]], sems[p],
                                 add=True)

                p3 = (p + 3) % 4

                @pl.when(j + 3 < CB)
                def _():
                    _wait_s(p3)
                    pltpu.async_copy(h_hbm.at[sidx.at[j + 3]],
                                     bufs.at[p3], semg[p3])

            return carry

        lax.fori_loop(1, CB // 4, _body, 0)
        j = CB - 1
        _wait_g(j, j % 4)
        pltpu.async_copy(bufs.at[j % 4], acc_sh.at[didx.at[j]], sems[j % 4],
                         add=True)
        for p in (1, 2, 3, 0):
            _wait_s(p)

    plsc.subcore_barrier()
    pltpu.sync_copy(acc_sh.at[pl.ds(base, PT)], out_hbm.at[c, pl.ds(base, PT)])


def _norm_from_parts(dref, which):
    deg = (dref[0, which] + dref[1, which])[:, 0:1]
    return lax.rsqrt(jnp.maximum(deg, 1.0))


def _prep_body(deg_ref, feat_ref, out_ref):
    out_ref[...] = feat_ref[...] * _norm_from_parts(deg_ref, 0)


def _prep_tc(deg_p, feat):
    return pl.pallas_call(
        _prep_body,
        grid=(N // RBT,),
        in_specs=[
            pl.BlockSpec((NC, 2, RBT, 16), lambda i: (0, 0, i, 0)),
            pl.BlockSpec((RBT, D), lambda i: (i, 0)),
        ],
        out_specs=pl.BlockSpec((RBT, D), lambda i: (i, 0)),
        out_shape=jax.ShapeDtypeStruct((N, D), jnp.float32),
    )(deg_p, feat)


def _mid_body(scale_out, aggp_ref, deg_ref, w_ref, b_ref, res_ref,
              *out_refs):
    agg = (aggp_ref[0] + aggp_ref[1]) * _norm_from_parts(deg_ref, 1)
    y = jnp.dot(agg, w_ref[...], preferred_element_type=jnp.float32) + b_ref[...]
    y = jnp.maximum(y, 0.0) + res_ref[...]
    out_refs[0][...] = y
    if scale_out:
        out_refs[1][...] = y * _norm_from_parts(deg_ref, 0)


def _mid_tc(aggp, deg_p, w, b, res, scale_out):
    n_out = 2 if scale_out else 1
    return pl.pallas_call(
        functools.partial(_mid_body, scale_out),
        grid=(N // RBT,),
        in_specs=[
            pl.BlockSpec((NC, RBT, D), lambda i: (0, i, 0)),
            pl.BlockSpec((NC, 2, RBT, 16), lambda i: (0, 0, i, 0)),
            pl.BlockSpec((D, D), lambda i: (0, 0)),
            pl.BlockSpec((1, D), lambda i: (0, 0)),
            pl.BlockSpec((RBT, D), lambda i: (i, 0)),
        ],
        out_specs=[pl.BlockSpec((RBT, D), lambda i: (i, 0))] * n_out,
        out_shape=[jax.ShapeDtypeStruct((N, D), jnp.float32)] * n_out,
    )(aggp, deg_p, w, b.reshape(1, D), res)


def kernel(feat, edge_index, etype, W1, b1, W2, b2, W3, b3):
    del etype
    ei5 = edge_index.reshape(2, NW, NBLK, CB, C)

    deg_p = _deg_kernel(ei5)

    h1s = _prep_tc(deg_p, feat)
    aggp = _agg_kernel(h1s, ei5)
    h1, h2s = _mid_tc(aggp, deg_p, W1, b1, feat, True)

    aggp = _agg_kernel(h2s, ei5)
    h2, h3s = _mid_tc(aggp, deg_p, W2, b2, h1, True)

    aggp = _agg_kernel(h3s, ei5)
    (h3,) = _mid_tc(aggp, deg_p, W3, b3, h2, False)

    return h3


# speedup vs baseline: 1.0187x; 1.0187x over previous
import functools

import jax
import jax.numpy as jnp
from jax import lax
from jax.experimental import pallas as pl
from jax.experimental.pallas import tpu as pltpu
from jax.experimental.pallas import tpu_sc as plsc

N = 10000
E = 320000
D = 128

NC = 2
NS = 16
NW = NC * NS
C = 80
NCH = 125
EP = NCH * C
NP = 10240
PT = NP // NS
RBT = 1000
CB = 25
NBLK = NCH // CB

_mesh = plsc.VectorSubcoreMesh(core_axis_name="c", subcore_axis_name="s")


@functools.partial(
    pl.kernel,
    out_type=(
        jax.ShapeDtypeStruct((NC, NP, 16), jnp.float32),
        jax.ShapeDtypeStruct((NC, NP, 16), jnp.float32),
    ),
    mesh=_mesh,
    scratch_types=[
        pltpu.VMEM((CB, C), jnp.int32),
        pltpu.VMEM((CB, C), jnp.int32),
        pltpu.VMEM((C, 16), jnp.float32),
        pltpu.VMEM((128, 16), jnp.float32),
        pltpu.VMEM_SHARED((NP, 16), jnp.float32),
        pltpu.VMEM_SHARED((NP, 16), jnp.float32),
        pltpu.SemaphoreType.DMA,
        pltpu.SemaphoreType.DMA,
    ],
    compiler_params=pltpu.CompilerParams(use_tc_tiling_on_sc=False),
)
def _deg_kernel(ei_hbm, osrc_hbm, odst_hbm,
                sidx, didx, ones, zb, dsrc_sh, ddst_sh, sem0, sem1):
    c = lax.axis_index("c")
    s = lax.axis_index("s")
    wid = s * NC + c

    one_hot = jnp.where(lax.iota(jnp.int32, 16) == 0, 1.0, 0.0).astype(jnp.float32)
    zvec = jnp.zeros((16,), jnp.float32)

    def _fill_ones(i, carry):
        ones[i, :] = one_hot
        return carry

    lax.fori_loop(0, C, _fill_ones, 0)

    def _fill_z(i, carry):
        zb[i, :] = zvec
        return carry

    lax.fori_loop(0, 128, _fill_z, 0)

    base = s * PT
    for k in range(PT // 128):
        pltpu.sync_copy(zb, dsrc_sh.at[pl.ds(base + k * 128, 128)])
        pltpu.sync_copy(zb, ddst_sh.at[pl.ds(base + k * 128, 128)])
    plsc.subcore_barrier()

    for blk in range(NBLK):
        pltpu.sync_copy(ei_hbm.at[0, wid, blk], sidx)
        pltpu.sync_copy(ei_hbm.at[1, wid, blk], didx)

        def _fire(j, carry):
            pltpu.async_copy(ones, dsrc_sh.at[sidx.at[j]], sem0, add=True)
            pltpu.async_copy(ones, ddst_sh.at[didx.at[j]], sem1, add=True)
            return carry

        lax.fori_loop(0, CB, _fire, 0)

        def _drain(j, carry):
            pltpu.make_async_copy(ones, dsrc_sh.at[sidx.at[j]], sem0).wait()
            pltpu.make_async_copy(ones, ddst_sh.at[didx.at[j]], sem1).wait()
            return carry

        lax.fori_loop(0, CB, _drain, 0)
    plsc.subcore_barrier()

    pltpu.sync_copy(dsrc_sh.at[pl.ds(base, PT)], osrc_hbm.at[c, pl.ds(base, PT)])
    pltpu.sync_copy(ddst_sh.at[pl.ds(base, PT)], odst_hbm.at[c, pl.ds(base, PT)])


@functools.partial(
    pl.kernel,
    out_type=jax.ShapeDtypeStruct((NC, NP, D), jnp.float32),
    mesh=_mesh,
    scratch_types=[
        pltpu.VMEM((CB, C), jnp.int32),
        pltpu.VMEM((CB, C), jnp.int32),
        pltpu.VMEM((4, C, D), jnp.float32),
        pltpu.VMEM_SHARED((NP, D), jnp.float32),
        pltpu.SemaphoreType.DMA,
        pltpu.SemaphoreType.DMA,
        pltpu.SemaphoreType.DMA,
        pltpu.SemaphoreType.DMA,
        pltpu.SemaphoreType.DMA,
        pltpu.SemaphoreType.DMA,
        pltpu.SemaphoreType.DMA,
        pltpu.SemaphoreType.DMA,
    ],
)
def _agg_kernel(h_hbm, ei_hbm, out_hbm,
                sidx, didx, bufs, acc_sh,
                g0, g1, g2, g3, s0, s1, s2, s3):
    c = lax.axis_index("c")
    s = lax.axis_index("s")
    wid = s * NC + c
    semg = (g0, g1, g2, g3)
    sems = (s0, s1, s2, s3)

    zvec = jnp.zeros((16,), jnp.float32)

    def _fill_z(i, carry):
        for k in range(D // 16):
            bufs[0, i, pl.ds(k * 16, 16)] = zvec
        return carry

    lax.fori_loop(0, C, _fill_z, 0)

    base = s * PT
    for k in range(PT // C):
        pltpu.sync_copy(bufs.at[0], acc_sh.at[pl.ds(base + k * C, C)])
    plsc.subcore_barrier()

    def _wait_g(j, p):
        pltpu.make_async_copy(h_hbm.at[sidx.at[j]], bufs.at[p], semg[p]).wait()

    def _wait_s(p):
        pltpu.make_async_copy(bufs.at[p], acc_sh.at[didx.at[0]], sems[p]).wait()

    for blk in range(NBLK):
        pltpu.sync_copy(ei_hbm.at[0, wid, blk], sidx)
        pltpu.sync_copy(ei_hbm.at[1, wid, blk], didx)

        for p in range(3):
            pltpu.async_copy(h_hbm.at[sidx.at[p]], bufs.at[p], semg[p])
        _wait_g(0, 0)
        pltpu.async_copy(bufs.at[0], acc_sh.at[didx.at[0]], sems[0], add=True)
        pltpu.async_copy(h_hbm.at[sidx.at[3]], bufs.at[3], semg[3])
        for j in range(1, 4):
            _wait_g(j, j)
            pltpu.async_copy(bufs.at[j], acc_sh.at[didx.at[j]], sems[j],
                             add=True)
            _wait_s(j - 1)
            pltpu.async_copy(h_hbm.at[sidx.at[j + 3]], bufs.at[j - 1],
                             semg[j - 1])

        def _body(t, carry):
            for p in range(4):
                j = t * 4 + p
                _wait_g(j, p)
                pltpu.async_copy(bufs.at[p], acc_sh.at[didx.at[j]], sems[p],
                                 add=True)

                p3 = (p + 3) % 4

                @pl.when(j + 3 < CB)
                def _():
                    _wait_s(p3)
                    pltpu.async_copy(h_hbm.at[sidx.at[j + 3]],
                                     bufs.at[p3], semg[p3])

            return carry

        lax.fori_loop(1, CB // 4, _body, 0)
        j = CB - 1
        _wait_g(j, j % 4)
        pltpu.async_copy(bufs.at[j % 4], acc_sh.at[didx.at[j]], sems[j % 4],
                         add=True)
        for p in (1, 2, 3, 0):
            _wait_s(p)

    plsc.subcore_barrier()
    pltpu.sync_copy(acc_sh.at[pl.ds(base, PT)], out_hbm.at[c, pl.ds(base, PT)])


def _norm_from_parts(dref):
    deg = (dref[0] + dref[1])[:, 0:1]
    return lax.rsqrt(jnp.maximum(deg, 1.0))


def _prep_body(dsrc_ref, feat_ref, out_ref):
    out_ref[...] = feat_ref[...] * _norm_from_parts(dsrc_ref)


def _prep_tc(dsrc_p, feat):
    return pl.pallas_call(
        _prep_body,
        grid=(N // RBT,),
        in_specs=[
            pl.BlockSpec((NC, RBT, 16), lambda i: (0, i, 0)),
            pl.BlockSpec((RBT, D), lambda i: (i, 0)),
        ],
        out_specs=pl.BlockSpec((RBT, D), lambda i: (i, 0)),
        out_shape=jax.ShapeDtypeStruct((N, D), jnp.float32),
    )(dsrc_p, feat)


def _mid_body(scale_out, aggp_ref, dsrc_ref, ddst_ref, w_ref, b_ref, res_ref,
              *out_refs):
    agg = (aggp_ref[0] + aggp_ref[1]) * _norm_from_parts(ddst_ref)
    y = jnp.dot(agg, w_ref[...], preferred_element_type=jnp.float32) + b_ref[...]
    y = jnp.maximum(y, 0.0) + res_ref[...]
    out_refs[0][...] = y
    if scale_out:
        out_refs[1][...] = y * _norm_from_parts(dsrc_ref)


def _mid_tc(aggp, dsrc_p, ddst_p, w, b, res, scale_out):
    n_out = 2 if scale_out else 1
    return pl.pallas_call(
        functools.partial(_mid_body, scale_out),
        grid=(N // RBT,),
        in_specs=[
            pl.BlockSpec((NC, RBT, D), lambda i: (0, i, 0)),
            pl.BlockSpec((NC, RBT, 16), lambda i: (0, i, 0)),
            pl.BlockSpec((NC, RBT, 16), lambda i: (0, i, 0)),
            pl.BlockSpec((D, D), lambda i: (0, 0)),
            pl.BlockSpec((1, D), lambda i: (0, 0)),
            pl.BlockSpec((RBT, D), lambda i: (i, 0)),
        ],
        out_specs=[pl.BlockSpec((RBT, D), lambda i: (i, 0))] * n_out,
        out_shape=[jax.ShapeDtypeStruct((N, D), jnp.float32)] * n_out,
    )(aggp, dsrc_p, ddst_p, w, b.reshape(1, D), res)


def kernel(feat, edge_index, etype, W1, b1, W2, b2, W3, b3):
    del etype
    ei5 = edge_index.reshape(2, NW, NBLK, CB, C)

    dsrc_p, ddst_p = _deg_kernel(ei5)

    h1s = _prep_tc(dsrc_p, feat)
    aggp = _agg_kernel(h1s, ei5)
    h1, h2s = _mid_tc(aggp, dsrc_p, ddst_p, W1, b1, feat, True)

    aggp = _agg_kernel(h2s, ei5)
    h2, h3s = _mid_tc(aggp, dsrc_p, ddst_p, W2, b2, h1, True)

    aggp = _agg_kernel(h3s, ei5)
    (h3,) = _mid_tc(aggp, dsrc_p, ddst_p, W3, b3, h2, False)

    return h3
